# baseline (speedup 1.0000x reference)
import functools

import jax
import jax.numpy as jnp
from jax import lax
from jax.experimental import pallas as pl
from jax.experimental.pallas import tpu as pltpu
from jax.experimental.pallas import tpu_sc as plsc

_NC = 2
_NS = 16
_NW = _NC * _NS


def _make_sc_broadcast(bsz, max_len, d_model):
    rows_per_w = max_len // _NW

    @functools.partial(
        pl.kernel,
        out_type=jax.ShapeDtypeStruct((bsz, max_len, d_model), jnp.float32),
        mesh=plsc.VectorSubcoreMesh(core_axis_name="c", subcore_axis_name="s"),
    )
    def k(table_hbm, out_hbm):
        wid = lax.axis_index("s") * _NC + lax.axis_index("c")
        base = wid * rows_per_w
        src = table_hbm.at[pl.ds(base, rows_per_w), :]
        for b in range(bsz):
            pltpu.sync_copy(src, out_hbm.at[b, pl.ds(base, rows_per_w), :])

    return k


def kernel(x, pos_embed_weight):
    bsz, seq_len = x.shape
    max_len, d_model = pos_embed_weight.shape
    return _make_sc_broadcast(bsz, max_len, d_model)(pos_embed_weight)


# speedup vs baseline: 51.2317x; 51.2317x over previous
import functools

import jax
import jax.numpy as jnp
from jax import lax
from jax.experimental import pallas as pl
from jax.experimental.pallas import tpu as pltpu
from jax.experimental.pallas import tpu_sc as plsc

_NC = 2
_NS = 16
_NW = _NC * _NS
_CHUNK = 64


def _make_sc_broadcast(bsz, max_len, d_model):
    rows_per_w = max_len // _NW
    n_chunks = rows_per_w // _CHUNK

    @functools.partial(
        pl.kernel,
        out_type=jax.ShapeDtypeStruct((bsz, max_len, d_model), jnp.float32),
        mesh=plsc.VectorSubcoreMesh(core_axis_name="c", subcore_axis_name="s"),
        scratch_types=[
            pltpu.VMEM((_CHUNK, d_model), jnp.float32),
            pltpu.VMEM((_CHUNK, d_model), jnp.float32),
            pltpu.SemaphoreType.DMA,
            pltpu.SemaphoreType.DMA,
            pltpu.SemaphoreType.DMA,
            pltpu.SemaphoreType.DMA,
        ],
    )
    def k(table_hbm, out_hbm, b0, b1, si0, si1, so0, so1):
        wid = lax.axis_index("s") * _NC + lax.axis_index("c")
        base = wid * rows_per_w
        bufs, sins, souts = (b0, b1), (si0, si1), (so0, so1)

        def in_desc(i):
            return pltpu.make_async_copy(
                table_hbm.at[pl.ds(base + i * _CHUNK, _CHUNK), :],
                bufs[i % 2], sins[i % 2])

        def out_desc(i, b):
            return pltpu.make_async_copy(
                bufs[i % 2],
                out_hbm.at[b, pl.ds(base + i * _CHUNK, _CHUNK), :],
                souts[i % 2])

        in_desc(0).start()
        for i in range(n_chunks):
            in_desc(i).wait()
            if i + 1 < n_chunks:
                if i >= 1:
                    for b in range(bsz):
                        out_desc(i - 1, b).wait()
                in_desc(i + 1).start()
            for b in range(bsz):
                out_desc(i, b).start()
        for i in (n_chunks - 2, n_chunks - 1):
            for b in range(bsz):
                out_desc(i, b).wait()

    return k


def kernel(x, pos_embed_weight):
    bsz, seq_len = x.shape
    max_len, d_model = pos_embed_weight.shape
    return _make_sc_broadcast(bsz, max_len, d_model)(pos_embed_weight)
